# Initial kernel scaffold; baseline (speedup 1.0000x reference)
import jax
import jax.numpy as jnp
from jax.experimental import pallas as pl


def kernel(x, edge_index, batch, Wg0, bg0, Wg1, bg1, Wg2, bg2, W1, b1, W2, b2, W3, b3):
    raise NotImplementedError("write your pallas kernel here")


# speedup vs baseline: 10.1753x; 10.1753x over previous
import functools

import jax
import jax.numpy as jnp
from jax import lax
from jax.experimental import pallas as pl
from jax.experimental.pallas import tpu as pltpu
from jax.experimental.pallas import tpu_sc as plsc

NC = 2
NS = 16
L = 16
MAX_RISK = 5.0

_F32 = jnp.float32


def _sc_mesh():
    return plsc.VectorSubcoreMesh(
        core_axis_name="c", subcore_axis_name="s", num_cores=NC, num_subcores=NS
    )


def _npad(n):
    return ((n + 1023) // 1024) * 1024


def _make_sc_pass(n, e, h, with_gather):
    nw = NC * NS
    ew = e // nw
    k = 80
    c = ew // k
    np_ = _npad(n)
    rs = np_ // NS
    rw = 64
    nr = rs // rw

    scratch = [
        pltpu.VMEM((k,), jnp.int32),
        pltpu.VMEM((rw,), jnp.int32),
        pltpu.VMEM((k, h), _F32),
        pltpu.VMEM((rw, h), _F32),
        pltpu.VMEM_SHARED((np_, h), _F32),
        pltpu.SemaphoreType.DMA,
    ]
    if with_gather:
        scratch.insert(0, pltpu.VMEM((k,), jnp.int32))

    def body(*refs):
        if with_gather:
            (y_hbm, src_hbm, dst_hbm, out_hbm,
             src_v, dst_v, rix_v, rows_v, buf_v, acc_sh, sem) = refs
        else:
            (dst_hbm, out_hbm,
             dst_v, rix_v, rows_v, buf_v, acc_sh, sem) = refs
        ci = lax.axis_index("c")
        si = lax.axis_index("s")
        wid = si * NC + ci

        def fill_zero(i, carry):
            for j in range(h // L):
                buf_v[i, pl.ds(j * L, L)] = jnp.zeros((L,), _F32)
            return carry

        lax.fori_loop(0, rw, fill_zero, 0)

        if not with_gather:
            def fill_ones(i, carry):
                for j in range(h // L):
                    rows_v[i, pl.ds(j * L, L)] = jnp.ones((L,), _F32)
                return carry

            lax.fori_loop(0, k, fill_ones, 0)

        def fill_rix(base):
            def fbody(j, carry):
                rix_v[pl.ds(j * L, L)] = base + j * L + lax.iota(jnp.int32, L)
                return carry
            lax.fori_loop(0, rw // L, fbody, 0)

        def zero_acc(r, carry):
            fill_rix(si * rs + r * rw)
            pltpu.sync_copy(buf_v, acc_sh.at[rix_v])
            return carry

        lax.fori_loop(0, nr, zero_acc, 0)
        plsc.subcore_barrier()

        def chunk(j, carry):
            base = wid * ew + j * k
            pltpu.sync_copy(dst_hbm.at[pl.ds(base, k)], dst_v)
            if with_gather:
                pltpu.sync_copy(src_hbm.at[pl.ds(base, k)], src_v)
                pltpu.async_copy(y_hbm.at[src_v], rows_v, sem).wait()
            pltpu.sync_copy(rows_v, acc_sh.at[dst_v], add=True)
            return carry

        lax.fori_loop(0, c, chunk, 0)
        plsc.subcore_barrier()

        def write_back(r, carry):
            off = si * rs + r * rw
            fill_rix(off)
            pltpu.async_copy(acc_sh.at[rix_v], buf_v, sem).wait()
            pltpu.sync_copy(buf_v, out_hbm.at[ci, pl.ds(off, rw)])
            return carry

        lax.fori_loop(0, nr, write_back, 0)

    return pl.kernel(
        body,
        out_type=jax.ShapeDtypeStruct((NC, np_, h), _F32),
        mesh=_sc_mesh(),
        scratch_types=scratch,
    )


@functools.partial(jax.jit, static_argnames=("n", "e", "h"))
def _sc_degrees(dst, *, n, e, h):
    return _make_sc_pass(n, e, h, with_gather=False)(dst)


@functools.partial(jax.jit, static_argnames=("n", "e", "h"))
def _sc_edge_sum(y, src, dst, *, n, e, h):
    return _make_sc_pass(n, e, h, with_gather=True)(y, src, dst)


def _dot(a, b):
    return lax.dot_general(
        a, b, (((1,), (0,)), ((), ())),
        preferred_element_type=_F32,
    )


def _tc_first(dinv2d, x, w0):
    n, d = x.shape
    h = w0.shape[1]

    def body(dinv_ref, x_ref, w_ref, dinvb_ref, y_ref):
        dinvb = jnp.broadcast_to(dinv_ref[...], (n, h))
        dinvb_ref[...] = dinvb
        y_ref[...] = dinvb * _dot(x_ref[...], w_ref[...])

    return pl.pallas_call(
        body,
        out_shape=[
            jax.ShapeDtypeStruct((n, h), _F32),
            jax.ShapeDtypeStruct((n, h), _F32),
        ],
    )(dinv2d, x, w0)


def _tc_mid(partials, y, dinvb, brow, wnext):
    n, h = y.shape

    def body(p_ref, y_ref, dinv_ref, b_ref, w_ref, ynext_ref):
        p = p_ref[...]
        hcur = jnp.maximum(
            (p[0, :n] + p[1, :n] + y_ref[...]) * dinv_ref[...] + b_ref[...],
            0.0)
        ynext_ref[...] = dinv_ref[...] * _dot(hcur, w_ref[...])

    return pl.pallas_call(
        body, out_shape=jax.ShapeDtypeStruct((n, h), _F32),
    )(partials, y, dinvb, brow, wnext)


def _tc_head(partials, y, dinvb, brow, batch2d, g,
             w1, b1row, w2, b2row, w3, b3row):
    n, h = y.shape

    def body(p_ref, y_ref, dinv_ref, b_ref, bat_ref,
             w1_ref, b1_ref, w2_ref, b2_ref, w3_ref, b3_ref, out_ref,
             gmp_ref):
        p = p_ref[...]
        hh = jnp.maximum(
            (p[0, :n] + p[1, :n] + y_ref[...]) * dinv_ref[...] + b_ref[...],
            0.0)
        bat = bat_ref[...]
        gids = lax.broadcasted_iota(jnp.int32, (n, g), 1)
        mask = (bat == gids).astype(_F32)
        sums = lax.dot_general(
            mask, hh, (((0,), (0,)), ((), ())),
            preferred_element_type=_F32, precision=lax.Precision.HIGHEST)
        cnts = lax.dot_general(
            mask, jnp.ones((n, 1), _F32), (((0,), (0,)), ((), ())),
            preferred_element_type=_F32, precision=lax.Precision.HIGHEST)

        def gstep(gi, carry):
            m = jnp.where(bat == gi, hh, -3.0e38)
            gmp_ref[pl.ds(gi, 1), :] = jnp.max(m, axis=0)[None, :]
            return carry

        lax.fori_loop(0, g, gstep, 0)
        gmp = jnp.where(cnts > 0.0, gmp_ref[...], 0.0)
        gap = sums / jnp.maximum(cnts, 1.0)
        z = jnp.concatenate([gmp, gap], axis=1)
        z = jnp.maximum(_dot(z, w1_ref[...]) + b1_ref[...], 0.0)
        z = jnp.maximum(_dot(z, w2_ref[...]) + b2_ref[...], 0.0)
        z = _dot(z, w3_ref[...]) + b3_ref[...]
        out_ref[...] = jnp.where(z > MAX_RISK, MAX_RISK, z)

    return pl.pallas_call(
        body, out_shape=jax.ShapeDtypeStruct((g, w3.shape[1]), _F32),
        scratch_shapes=[pltpu.VMEM((g, h), _F32)],
    )(partials, y, dinvb, brow, batch2d, w1, b1row, w2, b2row, w3, b3row)


def kernel(x, edge_index, batch, Wg0, bg0, Wg1, bg1, Wg2, bg2,
           W1, b1, W2, b2, W3, b3):
    n, d = x.shape
    e = edge_index.shape[1]
    h = Wg0.shape[1]
    g = 64

    src = edge_index[0]
    dst = edge_index[1]
    batch2d = batch[:, None]

    deg2 = _sc_degrees(dst, n=n, e=e, h=h)
    deg = deg2[0, :n, 0] + deg2[1, :n, 0] + 1.0
    dinv = jnp.where(deg > 0, deg ** -0.5, 0.0)
    dinvb, y0 = _tc_first(dinv[:, None], x, Wg0)

    p0 = _sc_edge_sum(y0, src, dst, n=n, e=e, h=h)
    y1 = _tc_mid(p0, y0, dinvb, bg0[None, :], Wg1)

    p1 = _sc_edge_sum(y1, src, dst, n=n, e=e, h=h)
    y2 = _tc_mid(p1, y1, dinvb, bg1[None, :], Wg2)

    p2 = _sc_edge_sum(y2, src, dst, n=n, e=e, h=h)
    return _tc_head(p2, y2, dinvb, bg2[None, :], batch2d, g,
                    W1, b1[None, :], W2, b2[None, :], W3, b3[None, :])


# speedup vs baseline: 12.2377x; 1.2027x over previous
import functools

import jax
import jax.numpy as jnp
from jax import lax
from jax.experimental import pallas as pl
from jax.experimental.pallas import tpu as pltpu
from jax.experimental.pallas import tpu_sc as plsc

NC = 2
NS = 16
L = 16
MAX_RISK = 5.0

_F32 = jnp.float32


def _sc_mesh():
    return plsc.VectorSubcoreMesh(
        core_axis_name="c", subcore_axis_name="s", num_cores=NC, num_subcores=NS
    )


def _npad(n):
    return ((n + 1023) // 1024) * 1024


def _make_sc_pass(n, e, h, with_gather):
    nw = NC * NS
    ew = e // nw
    k = 80
    c = ew // k
    np_ = _npad(n)
    rs = np_ // NS
    rw = 64
    nr = rs // rw

    scratch = [
        pltpu.VMEM((k,), jnp.int32),
        pltpu.VMEM((k,), jnp.int32),
        pltpu.VMEM((rw,), jnp.int32),
        pltpu.VMEM((k, h), _F32),
        pltpu.VMEM((k, h), _F32),
        pltpu.VMEM((rw, h), _F32),
        pltpu.VMEM_SHARED((np_, h), _F32),
        pltpu.SemaphoreType.DMA,
        pltpu.SemaphoreType.DMA,
        pltpu.SemaphoreType.DMA,
    ]
    if with_gather:
        scratch.insert(0, pltpu.VMEM((k,), jnp.int32))
        scratch.insert(1, pltpu.VMEM((k,), jnp.int32))
        scratch.append(pltpu.SemaphoreType.DMA)
        scratch.append(pltpu.SemaphoreType.DMA)

    def body(*refs):
        if with_gather:
            (y_hbm, src_hbm, dst_hbm, out_hbm,
             src_v0, src_v1, dst_v0, dst_v1, rix_v, rows_v0, rows_v1,
             buf_v, acc_sh, sem, ssem0, ssem1, gsem0, gsem1) = refs
        else:
            (dst_hbm, out_hbm,
             dst_v0, dst_v1, rix_v, rows_v0, rows_v1,
             buf_v, acc_sh, sem, ssem0, ssem1) = refs
        ci = lax.axis_index("c")
        si = lax.axis_index("s")
        wid = si * NC + ci

        def fill_zero(i, carry):
            for j in range(h // L):
                buf_v[i, pl.ds(j * L, L)] = jnp.zeros((L,), _F32)
            return carry

        lax.fori_loop(0, rw, fill_zero, 0)

        if not with_gather:
            def fill_ones(i, carry):
                for j in range(h // L):
                    rows_v0[i, pl.ds(j * L, L)] = jnp.ones((L,), _F32)
                return carry

            lax.fori_loop(0, k, fill_ones, 0)

        def fill_rix(base):
            def fbody(j, carry):
                rix_v[pl.ds(j * L, L)] = base + j * L + lax.iota(jnp.int32, L)
                return carry
            lax.fori_loop(0, rw // L, fbody, 0)

        def zero_acc(r, carry):
            fill_rix(si * rs + r * rw)
            pltpu.sync_copy(buf_v, acc_sh.at[rix_v])
            return carry

        lax.fori_loop(0, nr, zero_acc, 0)
        plsc.subcore_barrier()

        assert c % 2 == 1 and c >= 3

        if with_gather:
            def stage(cidx, src_v, dst_v, rows_v, gsem):
                base = wid * ew + cidx * k
                pltpu.sync_copy(src_hbm.at[pl.ds(base, k)], src_v)
                pltpu.sync_copy(dst_hbm.at[pl.ds(base, k)], dst_v)
                pltpu.async_copy(y_hbm.at[src_v], rows_v, gsem)

            stage(0, src_v0, dst_v0, rows_v0, gsem0)

            def chunk2(j, carry):
                c1 = 2 * j + 1
                cn = 2 * j + 2
                pltpu.make_async_copy(
                    y_hbm.at[src_v0], rows_v0, gsem0).wait()
                stage(c1, src_v1, dst_v1, rows_v1, gsem1)
                pltpu.async_copy(rows_v0, acc_sh.at[dst_v0], ssem0, add=True)
                pltpu.make_async_copy(
                    y_hbm.at[src_v1], rows_v1, gsem1).wait()
                pltpu.async_copy(rows_v1, acc_sh.at[dst_v1], ssem1, add=True)
                pltpu.make_async_copy(
                    rows_v0, acc_sh.at[dst_v0], ssem0).wait()
                stage(cn, src_v0, dst_v0, rows_v0, gsem0)
                pltpu.make_async_copy(
                    rows_v1, acc_sh.at[dst_v1], ssem1).wait()
                return carry

            lax.fori_loop(0, (c - 1) // 2, chunk2, 0)
            pltpu.make_async_copy(y_hbm.at[src_v0], rows_v0, gsem0).wait()
            pltpu.sync_copy(rows_v0, acc_sh.at[dst_v0], add=True)
        else:
            def staged(cidx, dst_v):
                base = wid * ew + cidx * k
                pltpu.sync_copy(dst_hbm.at[pl.ds(base, k)], dst_v)

            staged(0, dst_v0)

            def chunk2(j, carry):
                c1 = 2 * j + 1
                cn = 2 * j + 2
                pltpu.async_copy(rows_v0, acc_sh.at[dst_v0], ssem0, add=True)
                staged(c1, dst_v1)
                pltpu.async_copy(rows_v0, acc_sh.at[dst_v1], ssem1, add=True)
                pltpu.make_async_copy(
                    rows_v0, acc_sh.at[dst_v0], ssem0).wait()
                staged(cn, dst_v0)
                pltpu.make_async_copy(
                    rows_v0, acc_sh.at[dst_v1], ssem1).wait()
                return carry

            lax.fori_loop(0, (c - 1) // 2, chunk2, 0)
            pltpu.sync_copy(rows_v0, acc_sh.at[dst_v0], add=True)
        plsc.subcore_barrier()

        def write_back(r, carry):
            off = si * rs + r * rw
            fill_rix(off)
            pltpu.async_copy(acc_sh.at[rix_v], buf_v, sem).wait()
            pltpu.sync_copy(buf_v, out_hbm.at[ci, pl.ds(off, rw)])
            return carry

        lax.fori_loop(0, nr, write_back, 0)

    return pl.kernel(
        body,
        out_type=jax.ShapeDtypeStruct((NC, np_, h), _F32),
        mesh=_sc_mesh(),
        scratch_types=scratch,
    )


@functools.partial(jax.jit, static_argnames=("n", "e", "h"))
def _sc_degrees(dst, *, n, e, h):
    return _make_sc_pass(n, e, h, with_gather=False)(dst)


@functools.partial(jax.jit, static_argnames=("n", "e", "h"))
def _sc_edge_sum(y, src, dst, *, n, e, h):
    return _make_sc_pass(n, e, h, with_gather=True)(y, src, dst)


def _dot(a, b):
    return lax.dot_general(
        a, b, (((1,), (0,)), ((), ())),
        preferred_element_type=_F32,
    )


def _tc_first(dinv2d, x, w0):
    n, d = x.shape
    h = w0.shape[1]

    def body(dinv_ref, x_ref, w_ref, dinvb_ref, y_ref):
        dinvb = jnp.broadcast_to(dinv_ref[...], (n, h))
        dinvb_ref[...] = dinvb
        y_ref[...] = dinvb * _dot(x_ref[...], w_ref[...])

    return pl.pallas_call(
        body,
        out_shape=[
            jax.ShapeDtypeStruct((n, h), _F32),
            jax.ShapeDtypeStruct((n, h), _F32),
        ],
    )(dinv2d, x, w0)


def _tc_mid(partials, y, dinvb, brow, wnext):
    n, h = y.shape

    def body(p_ref, y_ref, dinv_ref, b_ref, w_ref, ynext_ref):
        p = p_ref[...]
        hcur = jnp.maximum(
            (p[0, :n] + p[1, :n] + y_ref[...]) * dinv_ref[...] + b_ref[...],
            0.0)
        ynext_ref[...] = dinv_ref[...] * _dot(hcur, w_ref[...])

    return pl.pallas_call(
        body, out_shape=jax.ShapeDtypeStruct((n, h), _F32),
    )(partials, y, dinvb, brow, wnext)


def _tc_head(partials, y, dinvb, brow, batch2d, g,
             w1, b1row, w2, b2row, w3, b3row):
    n, h = y.shape

    def body(p_ref, y_ref, dinv_ref, b_ref, bat_ref,
             w1_ref, b1_ref, w2_ref, b2_ref, w3_ref, b3_ref, out_ref,
             gmp_ref):
        p = p_ref[...]
        hh = jnp.maximum(
            (p[0, :n] + p[1, :n] + y_ref[...]) * dinv_ref[...] + b_ref[...],
            0.0)
        bat = bat_ref[...]
        gids = lax.broadcasted_iota(jnp.int32, (n, g), 1)
        mask = (bat == gids).astype(_F32)
        sums = lax.dot_general(
            mask, hh, (((0,), (0,)), ((), ())),
            preferred_element_type=_F32, precision=lax.Precision.HIGHEST)
        cnts = lax.dot_general(
            mask, jnp.ones((n, 1), _F32), (((0,), (0,)), ((), ())),
            preferred_element_type=_F32, precision=lax.Precision.HIGHEST)

        def gstep(gi, carry):
            m = jnp.where(bat == gi, hh, -3.0e38)
            gmp_ref[pl.ds(gi, 1), :] = jnp.max(m, axis=0)[None, :]
            return carry

        lax.fori_loop(0, g, gstep, 0)
        gmp = jnp.where(cnts > 0.0, gmp_ref[...], 0.0)
        gap = sums / jnp.maximum(cnts, 1.0)
        z = jnp.concatenate([gmp, gap], axis=1)
        z = jnp.maximum(_dot(z, w1_ref[...]) + b1_ref[...], 0.0)
        z = jnp.maximum(_dot(z, w2_ref[...]) + b2_ref[...], 0.0)
        z = _dot(z, w3_ref[...]) + b3_ref[...]
        out_ref[...] = jnp.where(z > MAX_RISK, MAX_RISK, z)

    return pl.pallas_call(
        body, out_shape=jax.ShapeDtypeStruct((g, w3.shape[1]), _F32),
        scratch_shapes=[pltpu.VMEM((g, h), _F32)],
    )(partials, y, dinvb, brow, batch2d, w1, b1row, w2, b2row, w3, b3row)


def kernel(x, edge_index, batch, Wg0, bg0, Wg1, bg1, Wg2, bg2,
           W1, b1, W2, b2, W3, b3):
    n, d = x.shape
    e = edge_index.shape[1]
    h = Wg0.shape[1]
    g = 64

    src = edge_index[0]
    dst = edge_index[1]
    batch2d = batch[:, None]

    deg2 = _sc_degrees(dst, n=n, e=e, h=h)
    deg = deg2[0, :n, 0] + deg2[1, :n, 0] + 1.0
    dinv = jnp.where(deg > 0, deg ** -0.5, 0.0)
    dinvb, y0 = _tc_first(dinv[:, None], x, Wg0)

    p0 = _sc_edge_sum(y0, src, dst, n=n, e=e, h=h)
    y1 = _tc_mid(p0, y0, dinvb, bg0[None, :], Wg1)

    p1 = _sc_edge_sum(y1, src, dst, n=n, e=e, h=h)
    y2 = _tc_mid(p1, y1, dinvb, bg1[None, :], Wg2)

    p2 = _sc_edge_sum(y2, src, dst, n=n, e=e, h=h)
    return _tc_head(p2, y2, dinvb, bg2[None, :], batch2d, g,
                    W1, b1[None, :], W2, b2[None, :], W3, b3[None, :])


# speedup vs baseline: 17.4260x; 1.4240x over previous
import functools

import jax
import jax.numpy as jnp
from jax import lax
from jax.experimental import pallas as pl
from jax.experimental.pallas import tpu as pltpu
from jax.experimental.pallas import tpu_sc as plsc

NC = 2
NS = 16
L = 16
MAX_RISK = 5.0

_F32 = jnp.float32


def _sc_mesh():
    return plsc.VectorSubcoreMesh(
        core_axis_name="c", subcore_axis_name="s", num_cores=NC, num_subcores=NS
    )


def _npad(n):
    return ((n + 1023) // 1024) * 1024


def _make_sc_pass(n, e, h, with_gather):
    nw = NC * NS
    ew = e // nw
    k = 80
    c = ew // k
    np_ = _npad(n)
    rs = np_ // NS
    rw = k
    nr = rs // rw
    assert rs % rw == 0 and c % 2 == 1 and c >= 3

    scratch = [
        pltpu.VMEM((c, k), jnp.int32),
        pltpu.VMEM((rw,), jnp.int32),
        pltpu.VMEM((k, h), _F32),
        pltpu.VMEM((k, h), _F32),
        pltpu.VMEM_SHARED((np_, h), _F32),
        pltpu.SemaphoreType.DMA,
        pltpu.SemaphoreType.DMA,
        pltpu.SemaphoreType.DMA,
    ]
    if with_gather:
        scratch.insert(0, pltpu.VMEM((ew,), jnp.int32))
        scratch.append(pltpu.SemaphoreType.DMA)
        scratch.append(pltpu.SemaphoreType.DMA)

    def body(*refs):
        if with_gather:
            (y_hbm, src_hbm, dst3_hbm, out_hbm,
             src_all, dst2_v, rix_v, rows_v0, rows_v1,
             acc_sh, sem, ssem0, ssem1, gsem0, gsem1) = refs
        else:
            (dst3_hbm, out_hbm,
             dst2_v, rix_v, rows_v0, rows_v1,
             acc_sh, sem, ssem0, ssem1) = refs
        ci = lax.axis_index("c")
        si = lax.axis_index("s")
        wid = si * NC + ci

        pltpu.sync_copy(dst3_hbm.at[wid], dst2_v)
        if with_gather:
            pltpu.sync_copy(src_hbm.at[pl.ds(wid * ew, ew)], src_all)

        zbuf = rows_v0 if with_gather else rows_v1

        def fill_zero(i, carry):
            for j in range(h // L):
                zbuf[i, pl.ds(j * L, L)] = jnp.zeros((L,), _F32)
            return carry

        lax.fori_loop(0, rw, fill_zero, 0)

        if not with_gather:
            def fill_ones(i, carry):
                for j in range(h // L):
                    rows_v0[i, pl.ds(j * L, L)] = jnp.ones((L,), _F32)
                return carry

            lax.fori_loop(0, k, fill_ones, 0)

        def fill_rix(base):
            def fbody(j, carry):
                rix_v[pl.ds(j * L, L)] = base + j * L + lax.iota(jnp.int32, L)
                return carry
            lax.fori_loop(0, rw // L, fbody, 0)

        def zero_acc(r, carry):
            fill_rix(si * rs + r * rw)
            pltpu.sync_copy(zbuf, acc_sh.at[rix_v])
            return carry

        lax.fori_loop(0, nr, zero_acc, 0)
        plsc.subcore_barrier()

        if with_gather:
            def gsrc(cidx):
                return y_hbm.at[src_all.at[pl.ds(cidx * k, k)]]

            pltpu.async_copy(gsrc(0), rows_v0, gsem0)

            def chunk2(j, carry):
                c0 = 2 * j
                c1 = 2 * j + 1
                cn = 2 * j + 2
                pltpu.make_async_copy(gsrc(c0), rows_v0, gsem0).wait()
                pltpu.async_copy(gsrc(c1), rows_v1, gsem1)
                pltpu.async_copy(
                    rows_v0, acc_sh.at[dst2_v.at[c0]], ssem0, add=True)
                pltpu.make_async_copy(gsrc(c1), rows_v1, gsem1).wait()
                pltpu.async_copy(
                    rows_v1, acc_sh.at[dst2_v.at[c1]], ssem1, add=True)
                pltpu.make_async_copy(
                    rows_v0, acc_sh.at[dst2_v.at[c0]], ssem0).wait()
                pltpu.async_copy(gsrc(cn), rows_v0, gsem0)
                pltpu.make_async_copy(
                    rows_v1, acc_sh.at[dst2_v.at[c1]], ssem1).wait()
                return carry

            lax.fori_loop(0, (c - 1) // 2, chunk2, 0)
            pltpu.make_async_copy(gsrc(c - 1), rows_v0, gsem0).wait()
            pltpu.sync_copy(rows_v0, acc_sh.at[dst2_v.at[c - 1]], add=True)
        else:
            def chunk2(j, carry):
                c0 = 2 * j
                c1 = 2 * j + 1
                pltpu.async_copy(
                    rows_v0, acc_sh.at[dst2_v.at[c0]], ssem0, add=True)
                pltpu.async_copy(
                    rows_v0, acc_sh.at[dst2_v.at[c1]], ssem1, add=True)
                pltpu.make_async_copy(
                    rows_v0, acc_sh.at[dst2_v.at[c0]], ssem0).wait()
                pltpu.make_async_copy(
                    rows_v0, acc_sh.at[dst2_v.at[c1]], ssem1).wait()
                return carry

            lax.fori_loop(0, (c - 1) // 2, chunk2, 0)
            pltpu.sync_copy(rows_v0, acc_sh.at[dst2_v.at[c - 1]], add=True)
        plsc.subcore_barrier()

        def write_back(r, carry):
            off = si * rs + r * rw
            fill_rix(off)
            pltpu.async_copy(acc_sh.at[rix_v], zbuf, sem).wait()
            pltpu.sync_copy(zbuf, out_hbm.at[ci, pl.ds(off, rw)])
            return carry

        lax.fori_loop(0, nr, write_back, 0)

    return pl.kernel(
        body,
        out_type=jax.ShapeDtypeStruct((NC, np_, h), _F32),
        mesh=_sc_mesh(),
        scratch_types=scratch,
    )


def _dst3(dst, e):
    return dst.reshape(NC * NS, -1, 80)


@functools.partial(jax.jit, static_argnames=("n", "e", "h"))
def _sc_degrees(dst, *, n, e, h):
    return _make_sc_pass(n, e, h, with_gather=False)(_dst3(dst, e))


@functools.partial(jax.jit, static_argnames=("n", "e", "h"))
def _sc_edge_sum(y, src, dst, *, n, e, h):
    return _make_sc_pass(n, e, h, with_gather=True)(y, src, _dst3(dst, e))


def _dot(a, b):
    return lax.dot_general(
        a, b, (((1,), (0,)), ((), ())),
        preferred_element_type=_F32,
    )


def _tc_first(dinv2d, x, w0):
    n, d = x.shape
    h = w0.shape[1]

    def body(dinv_ref, x_ref, w_ref, dinvb_ref, y_ref):
        dinvb = jnp.broadcast_to(dinv_ref[...], (n, h))
        dinvb_ref[...] = dinvb
        y_ref[...] = dinvb * _dot(x_ref[...], w_ref[...])

    return pl.pallas_call(
        body,
        out_shape=[
            jax.ShapeDtypeStruct((n, h), _F32),
            jax.ShapeDtypeStruct((n, h), _F32),
        ],
    )(dinv2d, x, w0)


def _tc_mid(partials, y, dinvb, brow, wnext):
    n, h = y.shape

    def body(p_ref, y_ref, dinv_ref, b_ref, w_ref, ynext_ref):
        p = p_ref[...]
        hcur = jnp.maximum(
            (p[0, :n] + p[1, :n] + y_ref[...]) * dinv_ref[...] + b_ref[...],
            0.0)
        ynext_ref[...] = dinv_ref[...] * _dot(hcur, w_ref[...])

    return pl.pallas_call(
        body, out_shape=jax.ShapeDtypeStruct((n, h), _F32),
    )(partials, y, dinvb, brow, wnext)


def _tc_head(partials, y, dinvb, brow, batch2d, g,
             w1, b1row, w2, b2row, w3, b3row):
    n, h = y.shape

    def body(p_ref, y_ref, dinv_ref, b_ref, bat_ref,
             w1_ref, b1_ref, w2_ref, b2_ref, w3_ref, b3_ref, out_ref,
             gmp_ref):
        p = p_ref[...]
        hh = jnp.maximum(
            (p[0, :n] + p[1, :n] + y_ref[...]) * dinv_ref[...] + b_ref[...],
            0.0)
        bat = bat_ref[...]
        gids = lax.broadcasted_iota(jnp.int32, (n, g), 1)
        mask = (bat == gids).astype(_F32)
        sums = lax.dot_general(
            mask, hh, (((0,), (0,)), ((), ())),
            preferred_element_type=_F32, precision=lax.Precision.HIGHEST)
        cnts = lax.dot_general(
            mask, jnp.ones((n, 1), _F32), (((0,), (0,)), ((), ())),
            preferred_element_type=_F32, precision=lax.Precision.HIGHEST)

        def gstep(gi, carry):
            m = jnp.where(bat == gi, hh, -3.0e38)
            gmp_ref[pl.ds(gi, 1), :] = jnp.max(m, axis=0)[None, :]
            return carry

        lax.fori_loop(0, g, gstep, 0)
        gmp = jnp.where(cnts > 0.0, gmp_ref[...], 0.0)
        gap = sums / jnp.maximum(cnts, 1.0)
        z = jnp.concatenate([gmp, gap], axis=1)
        z = jnp.maximum(_dot(z, w1_ref[...]) + b1_ref[...], 0.0)
        z = jnp.maximum(_dot(z, w2_ref[...]) + b2_ref[...], 0.0)
        z = _dot(z, w3_ref[...]) + b3_ref[...]
        out_ref[...] = jnp.where(z > MAX_RISK, MAX_RISK, z)

    return pl.pallas_call(
        body, out_shape=jax.ShapeDtypeStruct((g, w3.shape[1]), _F32),
        scratch_shapes=[pltpu.VMEM((g, h), _F32)],
    )(partials, y, dinvb, brow, batch2d, w1, b1row, w2, b2row, w3, b3row)


def kernel(x, edge_index, batch, Wg0, bg0, Wg1, bg1, Wg2, bg2,
           W1, b1, W2, b2, W3, b3):
    n, d = x.shape
    e = edge_index.shape[1]
    h = Wg0.shape[1]
    g = 64

    src = edge_index[0]
    dst = edge_index[1]
    batch2d = batch[:, None]

    deg2 = _sc_degrees(dst, n=n, e=e, h=h)
    deg = deg2[0, :n, 0] + deg2[1, :n, 0] + 1.0
    dinv = jnp.where(deg > 0, deg ** -0.5, 0.0)
    dinvb, y0 = _tc_first(dinv[:, None], x, Wg0)

    p0 = _sc_edge_sum(y0, src, dst, n=n, e=e, h=h)
    y1 = _tc_mid(p0, y0, dinvb, bg0[None, :], Wg1)

    p1 = _sc_edge_sum(y1, src, dst, n=n, e=e, h=h)
    y2 = _tc_mid(p1, y1, dinvb, bg1[None, :], Wg2)

    p2 = _sc_edge_sum(y2, src, dst, n=n, e=e, h=h)
    return _tc_head(p2, y2, dinvb, bg2[None, :], batch2d, g,
                    W1, b1[None, :], W2, b2[None, :], W3, b3[None, :])
